# baseline (speedup 1.0000x reference)
import jax
import jax.numpy as jnp
from jax import lax
from jax.experimental import pallas as pl
from jax.experimental.pallas import tpu as pltpu

_N = 4096
_D = 128
_DE = 256
_K = 4
_BR = 256
_NB = _N // _BR
_SLOPE = 0.01
_LOG2E = 1.4426950408889634


def _gat_body(x_ref, w_ref, a_ref, adj_ref, out_ref, h_ref, s_ref):
    step = pl.program_id(0)

    @pl.when(step == 0)
    def _prep():
        X = x_ref[:]
        lane = lax.broadcasted_iota(jnp.int32, (_N, _D), 1)
        denom_tile = jnp.where(lane == 0, 1.0, 0.0).astype(jnp.float32)
        for k in range(_K):
            Wk = w_ref[k]
            Hk = lax.dot_general(X, Wk, (((1,), (1,)), ((), ())),
                                 preferred_element_type=jnp.float32)
            h_ref[k] = jnp.concatenate([Hk, denom_tile], axis=1)
            a_nb = a_ref[k][:, :_D] * _LOG2E
            a_self = a_ref[k][:, _D:] * _LOG2E
            s_ref[pl.ds(k, 1), :] = lax.dot_general(
                a_nb, Hk, (((1,), (1,)), ((), ())),
                preferred_element_type=jnp.float32)
            s_ref[pl.ds(_K + k, 1), :] = lax.dot_general(
                a_self, Hk, (((1,), (1,)), ((), ())),
                preferred_element_type=jnp.float32)

    @pl.when(step > 0)
    def _attn():
        i = step - 1
        mask = adj_ref[:] != 0.0
        neg_inf = jnp.float32(-jnp.inf)
        for k in range(_K):
            He = h_ref[k]
            s_nb = s_ref[pl.ds(k, 1), :]
            s_self_row = s_ref[pl.ds(_K + k, 1), pl.ds(i * _BR, _BR)]
            s_self = lax.transpose(s_self_row, (1, 0))
            g = jnp.max(jnp.where(mask, s_nb, neg_inf), axis=1,
                        keepdims=True)
            t = s_self + g
            m = jnp.maximum(t, _SLOPE * t)
            scores = s_self + s_nb
            scores = jnp.maximum(scores, _SLOPE * scores)
            p = jnp.exp2(jnp.where(mask, scores, neg_inf) - m)
            o_ext = jnp.dot(p, He, preferred_element_type=jnp.float32)
            o = o_ext[:, :_D]
            denom = o_ext[:, _D:_D + 1]
            out_ref[:, k * _D:(k + 1) * _D] = o * (1.0 / denom)


@jax.jit
def kernel(X, adj, W, a):
    out = pl.pallas_call(
        _gat_body,
        grid=(1 + _NB,),
        in_specs=[
            pl.BlockSpec((_N, _D), lambda i: (0, 0)),
            pl.BlockSpec((_K, _D, _D), lambda i: (0, 0, 0)),
            pl.BlockSpec((_K, 1, 2 * _D), lambda i: (0, 0, 0)),
            pl.BlockSpec((_BR, _N), lambda i: (lax.max(i - 1, 0), 0)),
        ],
        out_specs=pl.BlockSpec((_BR, _K * _D), lambda i: (lax.max(i - 1, 0), 0)),
        out_shape=jax.ShapeDtypeStruct((_N, _K * _D), jnp.float32),
        scratch_shapes=[
            pltpu.VMEM((_K, _N, _DE), jnp.float32),
            pltpu.VMEM((8, _N), jnp.float32),
        ],
        compiler_params=pltpu.CompilerParams(
            dimension_semantics=("arbitrary",)),
    )(X, W, a, adj)
    return out


# speedup vs baseline: 1.0090x; 1.0090x over previous
import jax
import jax.numpy as jnp
from jax import lax
from jax.experimental import pallas as pl
from jax.experimental.pallas import tpu as pltpu

_N = 4096
_D = 128
_DE = 256
_K = 4
_BR = 256
_NB = _N // _BR
_SLOPE = 0.01
_LOG2E = 1.4426950408889634


def _gat_body(x_ref, w_ref, a_ref, adj_ref, out_ref, h_ref, s_ref):
    step = pl.program_id(0)

    @pl.when(step == 0)
    def _prep():
        X = x_ref[:]
        lane = lax.broadcasted_iota(jnp.int32, (_N, _D), 1)
        denom_tile = jnp.where(lane == 0, 1.0, 0.0).astype(jnp.float32)
        for k in range(_K):
            Wk = w_ref[k]
            Hk = lax.dot_general(X, Wk, (((1,), (1,)), ((), ())),
                                 preferred_element_type=jnp.float32)
            h_ref[k] = jnp.concatenate([Hk, denom_tile], axis=1)
            a_nb = a_ref[k][:, :_D] * _LOG2E
            a_self = a_ref[k][:, _D:] * _LOG2E
            s_ref[pl.ds(k, 1), :] = lax.dot_general(
                a_nb, Hk, (((1,), (1,)), ((), ())),
                preferred_element_type=jnp.float32)
            s_ref[pl.ds(_K + k, 1), :] = lax.dot_general(
                a_self, Hk, (((1,), (1,)), ((), ())),
                preferred_element_type=jnp.float32)

    @pl.when(step > 0)
    def _attn():
        i = step - 1
        mask = adj_ref[:] != 0.0
        neg_inf = jnp.float32(-jnp.inf)
        for k in range(_K):
            He = h_ref[k]
            s_nb = s_ref[pl.ds(k, 1), :]
            s_self_row = s_ref[pl.ds(_K + k, 1), pl.ds(i * _BR, _BR)]
            s_self = lax.transpose(s_self_row, (1, 0))
            scores = s_self + s_nb
            scores = jnp.maximum(scores, _SLOPE * scores)
            masked = jnp.where(mask, scores, neg_inf)
            m = jnp.max(masked, axis=1, keepdims=True)
            p = jnp.exp2(masked - m)
            o_ext = jnp.dot(p, He, preferred_element_type=jnp.float32)
            o = o_ext[:, :_D]
            denom = o_ext[:, _D:_D + 1]
            out_ref[:, k * _D:(k + 1) * _D] = o * (1.0 / denom)


@jax.jit
def kernel(X, adj, W, a):
    out = pl.pallas_call(
        _gat_body,
        grid=(1 + _NB,),
        in_specs=[
            pl.BlockSpec((_N, _D), lambda i: (0, 0)),
            pl.BlockSpec((_K, _D, _D), lambda i: (0, 0, 0)),
            pl.BlockSpec((_K, 1, 2 * _D), lambda i: (0, 0, 0)),
            pl.BlockSpec((_BR, _N), lambda i: (lax.max(i - 1, 0), 0)),
        ],
        out_specs=pl.BlockSpec((_BR, _K * _D), lambda i: (lax.max(i - 1, 0), 0)),
        out_shape=jax.ShapeDtypeStruct((_N, _K * _D), jnp.float32),
        scratch_shapes=[
            pltpu.VMEM((_K, _N, _DE), jnp.float32),
            pltpu.VMEM((8, _N), jnp.float32),
        ],
        compiler_params=pltpu.CompilerParams(
            dimension_semantics=("arbitrary",)),
    )(X, W, a, adj)
    return out
